# baseline (device time: 213844 ns/iter reference)
import jax
import jax.numpy as jnp
from jax import lax
from jax.experimental import pallas as pl
from jax.experimental.pallas import tpu as pltpu

MESH = pl.DeviceIdType.MESH
P = 1024
G = 4096
SB = 1024
HC = 384
H3C = 256
H3_OFF = 2 * HC
NH3 = 4
CR = 2048
NC = 8


def kernel(x):
    m, n = x.shape

    def body(x_ref, out_ref, xb, fbuf, pbuf, f_sems, p_sems, c_sems,
             inj_send, inj_recv, h3_send, h3_recv,
             sa_send, sa_recv, sb_send, sb_recv):
        my_x = lax.axis_index("x")
        my_y = lax.axis_index("y")
        my_z = lax.axis_index("z")
        partner = (1 - my_x, my_y, my_z)
        base = (1 - my_x) * m
        pbase = my_x * m

        barrier = pltpu.get_barrier_semaphore()

        def sig(dev):
            def f():
                pl.semaphore_signal(barrier, inc=1, device_id=dev,
                                    device_id_type=MESH)
            return f

        sig(partner)()
        c0 = jnp.int32(0)
        n_peers = jnp.int32(1)
        for cond, dev in (
            (my_y > 0, (my_x, jnp.maximum(my_y - 1, c0), my_z)),
            (my_y < 3, (my_x, jnp.minimum(my_y + 1, 3), my_z)),
            (my_z > 0, (my_x, my_y, jnp.maximum(my_z - 1, c0))),
            (my_z < 3, (my_x, my_y, jnp.minimum(my_z + 1, 3))),
        ):
            pl.when(cond)(sig(dev))
            n_peers = n_peers + cond.astype(jnp.int32)

        idx1 = my_y * 4 + my_z
        idx2 = my_z * 4 + my_y
        pld = []
        for k, idx in ((0, idx1), (1, idx2)):
            d = pltpu.make_async_copy(
                x_ref.at[pl.ds(idx * P, P), :], pbuf.at[k], p_sems.at[k])
            d.start()
            pld.append(d)

        loads = {}

        def load_chunk(k):
            d = pltpu.make_async_copy(
                x_ref.at[pl.ds(k * CR, CR), :], fbuf.at[k % 2],
                f_sems.at[k % 2])
            d.start()
            loads[k] = d

        load_chunk(0)
        pl.semaphore_wait(barrier, n_peers)

        for k, idx in ((0, idx1), (1, idx2)):
            pld[k].wait()
            xb[pl.ds(idx * P, P), :] = pbuf[k][...].astype(jnp.bfloat16)

        inj = []
        for k, (idx, col) in enumerate(((idx1, 0), (idx2, HC))):
            d = pltpu.make_async_remote_copy(
                src_ref=xb.at[pl.ds(idx * P, P), pl.ds(col, HC)],
                dst_ref=out_ref.at[pl.ds(pbase + idx * P, P), pl.ds(col, HC)],
                send_sem=inj_send.at[k],
                recv_sem=inj_recv.at[k],
                device_id=partner,
                device_id_type=MESH,
            )
            d.start()
            inj.append(d)

        def convert_chunk(k):
            loads[k].wait()
            if k + 1 < NC:
                load_chunk(k + 1)
            xb[pl.ds(k * CR, CR), :] = fbuf[k % 2][...].astype(jnp.bfloat16)

        h3 = []
        h3rows = m // NH3
        for j in range(NH3):
            h3.append(pltpu.make_async_remote_copy(
                src_ref=xb.at[pl.ds(j * h3rows, h3rows), pl.ds(H3_OFF, H3C)],
                dst_ref=out_ref.at[pl.ds(pbase + j * h3rows, h3rows),
                                   pl.ds(H3_OFF, H3C)],
                send_sem=h3_send.at[j],
                recv_sem=h3_recv.at[j],
                device_id=partner,
                device_id_type=MESH,
            ))

        def ax_coord(ax):
            return my_z if ax == "z" else my_y

        def dev_along(ax, c):
            c = jnp.clip(c, 0, 3)
            return (my_x, my_y, c) if ax == "z" else (my_x, c, my_z)

        def regionA(h, o):
            o = jnp.clip(o, 0, 3)
            row0 = base + ((my_y if h == 0 else my_z) * 4 + o) * P
            return out_ref.at[pl.ds(row0, P), pl.ds(h * HC, HC)]

        def regionB(h, o, sub):
            o = jnp.clip(o, 0, 3)
            row0 = base + o * G + sub * SB
            return out_ref.at[pl.ds(row0, SB), pl.ds(h * HC, HC)]

        def semA(h, st, t):
            return (h * 2 + st) * 3 + t

        def semB(h, st, t, sub):
            return ((h * 2 + st) * 3 + t) * 4 + sub

        sent = []

        def chain_roles(ax, st, t):
            c = ax_coord(ax)
            if st == 0:
                s_cond = (c >= t) & (c <= 2)
                s_o, s_tgt = c - t, c + 1
                r_cond = c >= t + 1
                r_o, r_src = c - 1 - t, c - 1
            else:
                s_cond = (c <= 3 - t) & (c >= 1)
                s_o, s_tgt = c + t, c - 1
                r_cond = c <= 2 - t
                r_o, r_src = c + 1 + t, c + 1
            return s_cond, s_o, s_tgt, r_cond, r_o, r_src

        def emitA(h, ax, st, t, role):
            s_cond, s_o, s_tgt, r_cond, r_o, r_src = chain_roles(ax, st, t)
            o, dev = (s_o, s_tgt) if role == "send" else (r_o, r_src)
            d = pltpu.make_async_remote_copy(
                src_ref=regionA(h, o), dst_ref=regionA(h, o),
                send_sem=sa_send.at[semA(h, st, t)],
                recv_sem=sa_recv.at[semA(h, st, t)],
                device_id=dev_along(ax, dev), device_id_type=MESH,
            )
            if role == "send":
                pl.when(s_cond)(lambda: d.start())
                sent.append((s_cond, d))
            else:
                pl.when(r_cond)(lambda: d.wait_recv())

        def emitB(h, ax, st, t, sub, role):
            s_cond, s_o, s_tgt, r_cond, r_o, r_src = chain_roles(ax, st, t)
            o, dev = (s_o, s_tgt) if role == "send" else (r_o, r_src)
            d = pltpu.make_async_remote_copy(
                src_ref=regionB(h, o, sub), dst_ref=regionB(h, o, sub),
                send_sem=sb_send.at[semB(h, st, t, sub)],
                recv_sem=sb_recv.at[semB(h, st, t, sub)],
                device_id=dev_along(ax, dev), device_id_type=MESH,
            )
            if role == "send":
                pl.when(s_cond)(lambda: d.start())
                sent.append((s_cond, d))
            else:
                pl.when(r_cond)(lambda: d.wait_recv())

        def own_copy(chunks):
            for k in chunks:
                d = pltpu.make_async_copy(
                    xb.at[pl.ds(k * CR, CR), :],
                    out_ref.at[pl.ds(pbase + k * CR, CR), :],
                    c_sems.at[k % 2])
                d.start()
                d.wait()

        A_specs = ((0, "z"), (1, "y"))
        for t in range(3):
            for h, ax in A_specs:
                if t == 0:
                    inj[h].wait_recv()
                emitA(h, ax, 0, t, "send")
                emitA(h, ax, 1, t, "send")
            convert_chunk(2 * t)
            convert_chunk(2 * t + 1)
            h3[t].start()
            for h, ax in A_specs:
                emitA(h, ax, 0, t, "recv")
                emitA(h, ax, 1, t, "recv")

        B_specs = ((0, "y"), (1, "z"))
        copy_plan = (range(0, 3), range(3, 6), range(6, 8))
        for t in range(3):
            for sub in range(4):
                for h, ax in B_specs:
                    emitB(h, ax, 0, t, sub, "send")
                    emitB(h, ax, 1, t, sub, "send")
            if t == 0:
                convert_chunk(6)
                convert_chunk(7)
                h3[3].start()
            own_copy(copy_plan[t])
            for sub in range(4):
                for h, ax in B_specs:
                    emitB(h, ax, 0, t, sub, "recv")
                    emitB(h, ax, 1, t, sub, "recv")

        for d in inj:
            d.wait_send()
        for d in h3:
            d.wait()
        for cond, d in sent:
            pl.when(cond)(lambda: d.wait_send())

    return pl.pallas_call(
        body,
        out_shape=jax.ShapeDtypeStruct((2 * m, n), jnp.bfloat16),
        in_specs=[pl.BlockSpec(memory_space=pl.ANY)],
        out_specs=pl.BlockSpec(memory_space=pl.ANY),
        scratch_shapes=[
            pltpu.VMEM((m, n), jnp.bfloat16),
            pltpu.VMEM((2, CR, n), jnp.float32),
            pltpu.VMEM((2, P, n), jnp.float32),
            pltpu.SemaphoreType.DMA((2,)),
            pltpu.SemaphoreType.DMA((2,)),
            pltpu.SemaphoreType.DMA((2,)),
            pltpu.SemaphoreType.DMA((2,)),
            pltpu.SemaphoreType.DMA((2,)),
            pltpu.SemaphoreType.DMA((NH3,)),
            pltpu.SemaphoreType.DMA((NH3,)),
            pltpu.SemaphoreType.DMA((12,)),
            pltpu.SemaphoreType.DMA((12,)),
            pltpu.SemaphoreType.DMA((48,)),
            pltpu.SemaphoreType.DMA((48,)),
        ],
        compiler_params=pltpu.CompilerParams(
            collective_id=0,
            vmem_limit_bytes=100 * 1024 * 1024,
        ),
    )(x)


# device time: 212349 ns/iter; 1.0070x vs baseline; 1.0070x over previous
import jax
import jax.numpy as jnp
from jax import lax
from jax.experimental import pallas as pl
from jax.experimental.pallas import tpu as pltpu

MESH = pl.DeviceIdType.MESH
P = 1024
G = 4096
SB = 2048
HC = 384
H3C = 256
H3_OFF = 2 * HC
NH3 = 4
CR = 2048
NC = 8


def kernel(x):
    m, n = x.shape

    def body(x_ref, out_ref, xb, fbuf, pbuf, f_sems, p_sems, c_sems,
             inj_send, inj_recv, h3_send, h3_recv,
             sa_send, sa_recv, sb_send, sb_recv):
        my_x = lax.axis_index("x")
        my_y = lax.axis_index("y")
        my_z = lax.axis_index("z")
        partner = (1 - my_x, my_y, my_z)
        base = (1 - my_x) * m
        pbase = my_x * m

        barrier = pltpu.get_barrier_semaphore()

        def sig(dev):
            def f():
                pl.semaphore_signal(barrier, inc=1, device_id=dev,
                                    device_id_type=MESH)
            return f

        sig(partner)()
        c0 = jnp.int32(0)
        n_peers = jnp.int32(1)
        for cond, dev in (
            (my_y > 0, (my_x, jnp.maximum(my_y - 1, c0), my_z)),
            (my_y < 3, (my_x, jnp.minimum(my_y + 1, 3), my_z)),
            (my_z > 0, (my_x, my_y, jnp.maximum(my_z - 1, c0))),
            (my_z < 3, (my_x, my_y, jnp.minimum(my_z + 1, 3))),
        ):
            pl.when(cond)(sig(dev))
            n_peers = n_peers + cond.astype(jnp.int32)

        idx1 = my_y * 4 + my_z
        idx2 = my_z * 4 + my_y
        pld = []
        for k, idx in ((0, idx1), (1, idx2)):
            d = pltpu.make_async_copy(
                x_ref.at[pl.ds(idx * P, P), :], pbuf.at[k], p_sems.at[k])
            d.start()
            pld.append(d)

        loads = {}

        def load_chunk(k):
            d = pltpu.make_async_copy(
                x_ref.at[pl.ds(k * CR, CR), :], fbuf.at[k % 2],
                f_sems.at[k % 2])
            d.start()
            loads[k] = d

        load_chunk(0)
        pl.semaphore_wait(barrier, n_peers)

        for k, idx in ((0, idx1), (1, idx2)):
            pld[k].wait()
            xb[pl.ds(idx * P, P), :] = pbuf[k][...].astype(jnp.bfloat16)

        inj = []
        for k, (idx, col) in enumerate(((idx1, 0), (idx2, HC))):
            d = pltpu.make_async_remote_copy(
                src_ref=xb.at[pl.ds(idx * P, P), pl.ds(col, HC)],
                dst_ref=out_ref.at[pl.ds(pbase + idx * P, P), pl.ds(col, HC)],
                send_sem=inj_send.at[k],
                recv_sem=inj_recv.at[k],
                device_id=partner,
                device_id_type=MESH,
            )
            d.start()
            inj.append(d)

        def convert_chunk(k):
            loads[k].wait()
            if k + 1 < NC:
                load_chunk(k + 1)
            xb[pl.ds(k * CR, CR), :] = fbuf[k % 2][...].astype(jnp.bfloat16)

        h3 = []
        h3rows = m // NH3
        for j in range(NH3):
            h3.append(pltpu.make_async_remote_copy(
                src_ref=xb.at[pl.ds(j * h3rows, h3rows), pl.ds(H3_OFF, H3C)],
                dst_ref=out_ref.at[pl.ds(pbase + j * h3rows, h3rows),
                                   pl.ds(H3_OFF, H3C)],
                send_sem=h3_send.at[j],
                recv_sem=h3_recv.at[j],
                device_id=partner,
                device_id_type=MESH,
            ))

        def ax_coord(ax):
            return my_z if ax == "z" else my_y

        def dev_along(ax, c):
            c = jnp.clip(c, 0, 3)
            return (my_x, my_y, c) if ax == "z" else (my_x, c, my_z)

        def regionA(h, o):
            o = jnp.clip(o, 0, 3)
            row0 = base + ((my_y if h == 0 else my_z) * 4 + o) * P
            return out_ref.at[pl.ds(row0, P), pl.ds(h * HC, HC)]

        def regionB(h, o, sub):
            o = jnp.clip(o, 0, 3)
            row0 = base + o * G + sub * SB
            return out_ref.at[pl.ds(row0, SB), pl.ds(h * HC, HC)]

        def semA(h, st, t):
            return (h * 2 + st) * 3 + t

        def semB(h, st, t, sub):
            return ((h * 2 + st) * 3 + t) * 2 + sub

        sent = []

        def chain_roles(ax, st, t):
            c = ax_coord(ax)
            if st == 0:
                s_cond = (c >= t) & (c <= 2)
                s_o, s_tgt = c - t, c + 1
                r_cond = c >= t + 1
                r_o, r_src = c - 1 - t, c - 1
            else:
                s_cond = (c <= 3 - t) & (c >= 1)
                s_o, s_tgt = c + t, c - 1
                r_cond = c <= 2 - t
                r_o, r_src = c + 1 + t, c + 1
            return s_cond, s_o, s_tgt, r_cond, r_o, r_src

        def emitA(h, ax, st, t, role):
            s_cond, s_o, s_tgt, r_cond, r_o, r_src = chain_roles(ax, st, t)
            o, dev = (s_o, s_tgt) if role == "send" else (r_o, r_src)
            d = pltpu.make_async_remote_copy(
                src_ref=regionA(h, o), dst_ref=regionA(h, o),
                send_sem=sa_send.at[semA(h, st, t)],
                recv_sem=sa_recv.at[semA(h, st, t)],
                device_id=dev_along(ax, dev), device_id_type=MESH,
            )
            if role == "send":
                pl.when(s_cond)(lambda: d.start())
                sent.append((s_cond, d))
            else:
                pl.when(r_cond)(lambda: d.wait_recv())

        def emitB(h, ax, st, t, sub, role):
            s_cond, s_o, s_tgt, r_cond, r_o, r_src = chain_roles(ax, st, t)
            o, dev = (s_o, s_tgt) if role == "send" else (r_o, r_src)
            d = pltpu.make_async_remote_copy(
                src_ref=regionB(h, o, sub), dst_ref=regionB(h, o, sub),
                send_sem=sb_send.at[semB(h, st, t, sub)],
                recv_sem=sb_recv.at[semB(h, st, t, sub)],
                device_id=dev_along(ax, dev), device_id_type=MESH,
            )
            if role == "send":
                pl.when(s_cond)(lambda: d.start())
                sent.append((s_cond, d))
            else:
                pl.when(r_cond)(lambda: d.wait_recv())

        def own_copy(chunks):
            for k in chunks:
                d = pltpu.make_async_copy(
                    xb.at[pl.ds(k * CR, CR), :],
                    out_ref.at[pl.ds(pbase + k * CR, CR), :],
                    c_sems.at[k % 2])
                d.start()
                d.wait()

        A_specs = ((0, "z"), (1, "y"))
        for t in range(3):
            for h, ax in A_specs:
                if t == 0:
                    inj[h].wait_recv()
                for st in (0, 1):
                    if t > 0:
                        emitA(h, ax, st, t - 1, "recv")
                    emitA(h, ax, st, t, "send")
            convert_chunk(2 * t)
            convert_chunk(2 * t + 1)
            h3[t].start()
        for h, ax in A_specs:
            for st in (0, 1):
                emitA(h, ax, st, 2, "recv")

        B_specs = ((0, "y"), (1, "z"))
        copy_plan = (range(0, 4), range(4, 8), range(8, 8))
        for t in range(3):
            for h, ax in B_specs:
                for st in (0, 1):
                    for sub in range(2):
                        if t > 0:
                            emitB(h, ax, st, t - 1, sub, "recv")
                        emitB(h, ax, st, t, sub, "send")
            if t == 0:
                convert_chunk(6)
                convert_chunk(7)
                h3[3].start()
            own_copy(copy_plan[t])
        for h, ax in B_specs:
            for st in (0, 1):
                for sub in range(2):
                    emitB(h, ax, st, 2, sub, "recv")

        for d in inj:
            d.wait_send()
        for d in h3:
            d.wait()
        for cond, d in sent:
            pl.when(cond)(lambda: d.wait_send())

    return pl.pallas_call(
        body,
        out_shape=jax.ShapeDtypeStruct((2 * m, n), jnp.bfloat16),
        in_specs=[pl.BlockSpec(memory_space=pl.ANY)],
        out_specs=pl.BlockSpec(memory_space=pl.ANY),
        scratch_shapes=[
            pltpu.VMEM((m, n), jnp.bfloat16),
            pltpu.VMEM((2, CR, n), jnp.float32),
            pltpu.VMEM((2, P, n), jnp.float32),
            pltpu.SemaphoreType.DMA((2,)),
            pltpu.SemaphoreType.DMA((2,)),
            pltpu.SemaphoreType.DMA((2,)),
            pltpu.SemaphoreType.DMA((2,)),
            pltpu.SemaphoreType.DMA((2,)),
            pltpu.SemaphoreType.DMA((NH3,)),
            pltpu.SemaphoreType.DMA((NH3,)),
            pltpu.SemaphoreType.DMA((12,)),
            pltpu.SemaphoreType.DMA((12,)),
            pltpu.SemaphoreType.DMA((24,)),
            pltpu.SemaphoreType.DMA((24,)),
        ],
        compiler_params=pltpu.CompilerParams(
            collective_id=0,
            vmem_limit_bytes=100 * 1024 * 1024,
        ),
    )(x)


# device time: 212287 ns/iter; 1.0073x vs baseline; 1.0003x over previous
import jax
import jax.numpy as jnp
from jax import lax
from jax.experimental import pallas as pl
from jax.experimental.pallas import tpu as pltpu

MESH = pl.DeviceIdType.MESH
P = 1024
G = 4096
SB = 2048
HC = 384
H3C = 256
H3_OFF = 2 * HC
NH3 = 4
CR = 2048
NC = 8


def kernel(x):
    m, n = x.shape

    def body(x_ref, _donated_ref, out_ref, xb, fbuf, pbuf, f_sems, p_sems, c_sems,
             inj_send, inj_recv, h3_send, h3_recv,
             sa_send, sa_recv, sb_send, sb_recv):
        my_x = lax.axis_index("x")
        my_y = lax.axis_index("y")
        my_z = lax.axis_index("z")
        partner = (1 - my_x, my_y, my_z)
        base = (1 - my_x) * m
        pbase = my_x * m

        barrier = pltpu.get_barrier_semaphore()

        def sig(dev):
            def f():
                pl.semaphore_signal(barrier, inc=1, device_id=dev,
                                    device_id_type=MESH)
            return f

        sig(partner)()
        c0 = jnp.int32(0)
        n_peers = jnp.int32(1)
        for cond, dev in (
            (my_y > 0, (my_x, jnp.maximum(my_y - 1, c0), my_z)),
            (my_y < 3, (my_x, jnp.minimum(my_y + 1, 3), my_z)),
            (my_z > 0, (my_x, my_y, jnp.maximum(my_z - 1, c0))),
            (my_z < 3, (my_x, my_y, jnp.minimum(my_z + 1, 3))),
        ):
            pl.when(cond)(sig(dev))
            n_peers = n_peers + cond.astype(jnp.int32)

        idx1 = my_y * 4 + my_z
        idx2 = my_z * 4 + my_y
        pld = []
        for k, idx in ((0, idx1), (1, idx2)):
            d = pltpu.make_async_copy(
                x_ref.at[pl.ds(idx * P, P), :], pbuf.at[k], p_sems.at[k])
            d.start()
            pld.append(d)

        loads = {}

        def load_chunk(k):
            d = pltpu.make_async_copy(
                x_ref.at[pl.ds(k * CR, CR), :], fbuf.at[k % 2],
                f_sems.at[k % 2])
            d.start()
            loads[k] = d

        load_chunk(0)
        pl.semaphore_wait(barrier, n_peers)

        for k, idx in ((0, idx1), (1, idx2)):
            pld[k].wait()
            xb[pl.ds(idx * P, P), :] = pbuf[k][...].astype(jnp.bfloat16)

        inj = []
        for k, (idx, col) in enumerate(((idx1, 0), (idx2, HC))):
            d = pltpu.make_async_remote_copy(
                src_ref=xb.at[pl.ds(idx * P, P), pl.ds(col, HC)],
                dst_ref=out_ref.at[pl.ds(pbase + idx * P, P), pl.ds(col, HC)],
                send_sem=inj_send.at[k],
                recv_sem=inj_recv.at[k],
                device_id=partner,
                device_id_type=MESH,
            )
            d.start()
            inj.append(d)

        def convert_chunk(k):
            loads[k].wait()
            if k + 1 < NC:
                load_chunk(k + 1)
            xb[pl.ds(k * CR, CR), :] = fbuf[k % 2][...].astype(jnp.bfloat16)

        h3 = []
        h3rows = m // NH3
        for j in range(NH3):
            h3.append(pltpu.make_async_remote_copy(
                src_ref=xb.at[pl.ds(j * h3rows, h3rows), pl.ds(H3_OFF, H3C)],
                dst_ref=out_ref.at[pl.ds(pbase + j * h3rows, h3rows),
                                   pl.ds(H3_OFF, H3C)],
                send_sem=h3_send.at[j],
                recv_sem=h3_recv.at[j],
                device_id=partner,
                device_id_type=MESH,
            ))

        def ax_coord(ax):
            return my_z if ax == "z" else my_y

        def dev_along(ax, c):
            c = jnp.clip(c, 0, 3)
            return (my_x, my_y, c) if ax == "z" else (my_x, c, my_z)

        def regionA(h, o):
            o = jnp.clip(o, 0, 3)
            row0 = base + ((my_y if h == 0 else my_z) * 4 + o) * P
            return out_ref.at[pl.ds(row0, P), pl.ds(h * HC, HC)]

        def regionB(h, o, sub):
            o = jnp.clip(o, 0, 3)
            row0 = base + o * G + sub * SB
            return out_ref.at[pl.ds(row0, SB), pl.ds(h * HC, HC)]

        def semA(h, st, t):
            return (h * 2 + st) * 3 + t

        def semB(h, st, t, sub):
            return ((h * 2 + st) * 3 + t) * 2 + sub

        sent = []

        def chain_roles(ax, st, t):
            c = ax_coord(ax)
            if st == 0:
                s_cond = (c >= t) & (c <= 2)
                s_o, s_tgt = c - t, c + 1
                r_cond = c >= t + 1
                r_o, r_src = c - 1 - t, c - 1
            else:
                s_cond = (c <= 3 - t) & (c >= 1)
                s_o, s_tgt = c + t, c - 1
                r_cond = c <= 2 - t
                r_o, r_src = c + 1 + t, c + 1
            return s_cond, s_o, s_tgt, r_cond, r_o, r_src

        def emitA(h, ax, st, t, role):
            s_cond, s_o, s_tgt, r_cond, r_o, r_src = chain_roles(ax, st, t)
            o, dev = (s_o, s_tgt) if role == "send" else (r_o, r_src)
            d = pltpu.make_async_remote_copy(
                src_ref=regionA(h, o), dst_ref=regionA(h, o),
                send_sem=sa_send.at[semA(h, st, t)],
                recv_sem=sa_recv.at[semA(h, st, t)],
                device_id=dev_along(ax, dev), device_id_type=MESH,
            )
            if role == "send":
                pl.when(s_cond)(lambda: d.start())
                sent.append((s_cond, d))
            else:
                pl.when(r_cond)(lambda: d.wait_recv())

        def emitB(h, ax, st, t, sub, role):
            s_cond, s_o, s_tgt, r_cond, r_o, r_src = chain_roles(ax, st, t)
            o, dev = (s_o, s_tgt) if role == "send" else (r_o, r_src)
            d = pltpu.make_async_remote_copy(
                src_ref=regionB(h, o, sub), dst_ref=regionB(h, o, sub),
                send_sem=sb_send.at[semB(h, st, t, sub)],
                recv_sem=sb_recv.at[semB(h, st, t, sub)],
                device_id=dev_along(ax, dev), device_id_type=MESH,
            )
            if role == "send":
                pl.when(s_cond)(lambda: d.start())
                sent.append((s_cond, d))
            else:
                pl.when(r_cond)(lambda: d.wait_recv())

        def own_copy(chunks):
            for k in chunks:
                d = pltpu.make_async_copy(
                    xb.at[pl.ds(k * CR, CR), :],
                    out_ref.at[pl.ds(pbase + k * CR, CR), :],
                    c_sems.at[k % 2])
                d.start()
                d.wait()

        A_specs = ((0, "z"), (1, "y"))
        for t in range(3):
            for h, ax in A_specs:
                if t == 0:
                    inj[h].wait_recv()
                for st in (0, 1):
                    if t > 0:
                        emitA(h, ax, st, t - 1, "recv")
                    emitA(h, ax, st, t, "send")
            convert_chunk(2 * t)
            convert_chunk(2 * t + 1)
            h3[t].start()
        for h, ax in A_specs:
            for st in (0, 1):
                emitA(h, ax, st, 2, "recv")

        B_specs = ((0, "y"), (1, "z"))
        copy_plan = (range(0, 4), range(4, 8), range(8, 8))
        for t in range(3):
            for h, ax in B_specs:
                for st in (0, 1):
                    for sub in range(2):
                        if t > 0:
                            emitB(h, ax, st, t - 1, sub, "recv")
                        emitB(h, ax, st, t, sub, "send")
            if t == 0:
                convert_chunk(6)
                convert_chunk(7)
                h3[3].start()
            own_copy(copy_plan[t])
        for h, ax in B_specs:
            for st in (0, 1):
                for sub in range(2):
                    emitB(h, ax, st, 2, sub, "recv")

        for d in inj:
            d.wait_send()
        for d in h3:
            d.wait()
        for cond, d in sent:
            pl.when(cond)(lambda: d.wait_send())

    out_buf = pl.empty((2 * m, n), jnp.bfloat16)
    return pl.pallas_call(
        body,
        out_shape=jax.ShapeDtypeStruct((2 * m, n), jnp.bfloat16),
        in_specs=[pl.BlockSpec(memory_space=pl.ANY),
                  pl.BlockSpec(memory_space=pl.ANY)],
        out_specs=pl.BlockSpec(memory_space=pl.ANY),
        input_output_aliases={1: 0},
        scratch_shapes=[
            pltpu.VMEM((m, n), jnp.bfloat16),
            pltpu.VMEM((2, CR, n), jnp.float32),
            pltpu.VMEM((2, P, n), jnp.float32),
            pltpu.SemaphoreType.DMA((2,)),
            pltpu.SemaphoreType.DMA((2,)),
            pltpu.SemaphoreType.DMA((2,)),
            pltpu.SemaphoreType.DMA((2,)),
            pltpu.SemaphoreType.DMA((2,)),
            pltpu.SemaphoreType.DMA((NH3,)),
            pltpu.SemaphoreType.DMA((NH3,)),
            pltpu.SemaphoreType.DMA((12,)),
            pltpu.SemaphoreType.DMA((12,)),
            pltpu.SemaphoreType.DMA((24,)),
            pltpu.SemaphoreType.DMA((24,)),
        ],
        compiler_params=pltpu.CompilerParams(
            collective_id=0,
            vmem_limit_bytes=100 * 1024 * 1024,
        ),
    )(x, out_buf)
